# baseline (speedup 1.0000x reference)
import jax
import jax.numpy as jnp
from jax import lax
from jax.experimental import pallas as pl
from jax.experimental.pallas import tpu as pltpu
from jax.experimental.pallas import tpu_sc as plsc

_NC = 2
_NS = 16
_NW = _NC * _NS
_LANES = 16

_BLK = 128


def _build_relayout(V, D, tc_tiling=True):
    full_cols = V // 128
    tail = V - full_cols * 128
    per_w = -(-full_cols // _NW)
    DT = D // 8

    mesh = plsc.VectorSubcoreMesh(core_axis_name="c", subcore_axis_name="s")

    @pl.kernel(
        out_type=jax.ShapeDtypeStruct((V * D,), jnp.float32),
        mesh=mesh,
        compiler_params=pltpu.CompilerParams(
            needs_layout_passes=False, use_tc_tiling_on_sc=tc_tiling),
        scratch_types=[
            pltpu.VMEM((2, D, 128), jnp.float32),
            pltpu.VMEM((2, 128 * D), jnp.float32),
            pltpu.SemaphoreType.DMA,
            pltpu.SemaphoreType.DMA,
            pltpu.SemaphoreType.DMA,
            pltpu.SemaphoreType.DMA,
        ],
    )
    def relayout_kernel(t2_hbm, tail_hbm, lin_hbm, tile_v, lin_v,
                        isem_a, isem_b, osem_a, osem_b):
        wid = lax.axis_index("s") * _NC + lax.axis_index("c")
        base = wid * per_w
        iota = lax.broadcasted_iota(jnp.int32, (_LANES,), 0)

        isems = [isem_a, isem_b]
        osems = [osem_a, osem_b]

        def col(t):
            return lax.rem(base + t, full_cols)

        def fetch(t, par):
            for i in range(DT):
                pltpu.async_copy(
                    t2_hbm.at[pl.ds(i * 8, 8), pl.ds(col(t) * 128, 128)],
                    tile_v.at[par, pl.ds(i * 8, 8)], isems[par])

        def process(t, par, first):
            pltpu.make_async_copy(
                t2_hbm.at[:, pl.ds(0, 128)], tile_v.at[par],
                isems[par]).wait()
            if not first:
                pltpu.make_async_copy(
                    lin_hbm.at[pl.ds(0, 128 * D)], lin_v.at[par],
                    osems[par]).wait()

            @pl.loop(0, 128 * (D // _LANES), unroll=16)
            def _tr(j):
                r = j >> 1
                dh = j & (D // _LANES - 1)
                v = plsc.load_gather(
                    tile_v.at[par],
                    [dh * _LANES + iota,
                     jnp.full((_LANES,), 0, jnp.int32) + r])
                lin_v[par, pl.ds(r * D + dh * _LANES, _LANES)] = v

            pltpu.async_copy(
                lin_v.at[par],
                lin_hbm.at[pl.ds(col(t) * 128 * D, 128 * D)],
                osems[par])

        fetch(0, 0)
        fetch(1, 1)
        process(0, 0, True)
        fetch(2, 0)
        process(1, 1, True)

        @pl.loop(0, (per_w - 3) // 2)
        def _cols(u):
            t = 3 + 2 * u
            fetch(t, 1)
            process(t - 1, 0, False)
            fetch(t + 1, 0)
            process(t, 1, False)

        process(per_w - 1, (per_w - 1) % 2, False)
        for par in range(2):
            pltpu.make_async_copy(
                lin_hbm.at[pl.ds(0, 128 * D)], lin_v.at[par],
                osems[par]).wait()

        if tail:
            @pl.when(wid == _NW - 1)
            def _tail():
                pltpu.sync_copy(tail_hbm, lin_v.at[0, pl.ds(0, tail * D)])
                pltpu.sync_copy(
                    lin_v.at[0, pl.ds(0, tail * D)],
                    lin_hbm.at[pl.ds(full_cols * 128 * D, tail * D)])

    return relayout_kernel


def _build_gather(S, H, D, n_off_pad):
    NSH = S * H
    blocks = NSH * 8
    assert blocks % _NW == 0
    per_w = blocks // _NW
    n_ids_w = per_w * _BLK
    nvec = n_ids_w // _LANES
    DT = D // 8

    mesh = plsc.VectorSubcoreMesh(core_axis_name="c", subcore_axis_name="s")

    @pl.kernel(
        out_type=jax.ShapeDtypeStruct((NSH, DT, 8, 8, 128), jnp.float32),
        mesh=mesh,
        compiler_params=pltpu.CompilerParams(
            needs_layout_passes=False, use_tc_tiling_on_sc=False),
        scratch_types=[
            pltpu.VMEM((n_off_pad,), jnp.int32),
            pltpu.VMEM((n_ids_w,), jnp.int32),
            pltpu.VMEM((2, _BLK, D), jnp.float32),
            pltpu.VMEM((2, DT, 8, 128), jnp.float32),
            pltpu.SemaphoreType.DMA,
            pltpu.SemaphoreType.DMA,
            pltpu.SemaphoreType.DMA,
            pltpu.SemaphoreType.DMA,
        ],
    )
    def gather_kernel(ids_hbm, offs_hbm, table_hbm, o5,
                      offs_v, idx_v, rows_v, tblk_v,
                      gsem_a, gsem_b, osem_a, osem_b):
        wid = lax.axis_index("s") * _NC + lax.axis_index("c")
        base = wid * per_w

        pltpu.sync_copy(offs_hbm, offs_v)
        pltpu.sync_copy(ids_hbm.at[pl.ds(base * _BLK, n_ids_w)], idx_v)
        iota = lax.broadcasted_iota(jnp.int32, (_LANES,), 0)

        @pl.loop(0, nvec, unroll=8)
        def _add_off(j):
            h = lax.rem((base + (j >> 3)) >> 3, H)
            off = plsc.load_gather(offs_v, [jnp.full((_LANES,), 0, jnp.int32) + h])
            sl = pl.ds(j * _LANES, _LANES)
            idx_v[sl] = idx_v[sl] + off

        gsems = [gsem_a, gsem_b]
        osems = [osem_a, osem_b]

        def issue(g, par):
            pltpu.async_copy(
                table_hbm.at[idx_v.at[pl.ds(g * _BLK, _BLK)]],
                rows_v.at[par], gsems[par])

        def finish(g, par, first):
            pltpu.make_async_copy(
                table_hbm.at[pl.ds(0, _BLK)], rows_v.at[par],
                gsems[par]).wait()
            if not first:
                for ti in range(DT):
                    pltpu.make_async_copy(
                        o5.at[0, 0, 0], tblk_v.at[par, ti],
                        osems[par]).wait()
            @pl.loop(0, (_BLK * D) // (_LANES * _LANES), unroll=16)
            def _tr(j):
                ti = j >> 6
                sub = (j >> 3) & 7
                kk = j & 7
                lanes = kk * _LANES + iota
                dims = jnp.full((_LANES,), 0, jnp.int32) + (ti * 8 + sub)
                v = plsc.load_gather(rows_v.at[par], [lanes, dims])
                tblk_v[par, ti, sub, pl.ds(kk * _LANES, _LANES)] = v

            B = base + g
            sh = B >> 3
            tj = lax.rem(B, 8)
            for ti in range(DT):
                pltpu.async_copy(
                    tblk_v.at[par, ti], o5.at[sh, ti, tj], osems[par])

        issue(0, 0)
        issue(1, 1)
        finish(0, 0, True)
        issue(2, 0)
        finish(1, 1, True)

        @pl.loop(0, (per_w - 3) // 2)
        def _blocks(t):
            g = 3 + 2 * t
            issue(g, 1)
            finish(g - 1, 0, False)
            issue(g + 1, 0)
            finish(g, 1, False)

        finish(per_w - 1, 0, False)
        for par in range(2):
            for ti in range(DT):
                pltpu.make_async_copy(
                    o5.at[0, 0, 0], tblk_v.at[par, ti], osems[par]).wait()

    return gather_kernel


def kernel(head_ids, offsets, table):
    B, S, H = head_ids.shape
    V, D = table.shape
    n_off_pad = 128
    ids_t = jnp.transpose(head_ids, (1, 2, 0)).reshape(-1).astype(jnp.int32)
    offs = jnp.zeros((n_off_pad,), jnp.int32).at[:H].set(
        offsets.astype(jnp.int32))
    lin = table.reshape(V * D)
    o5 = _build_gather(S, H, D, n_off_pad)(ids_t, offs, lin.reshape(V, D))
    out = (o5.reshape(S, H, D // 8, 8, 8, 128)
           .transpose(3, 5, 0, 1, 2, 4)
           .reshape(B, S, H, D))
    return out
